# baseline (speedup 1.0000x reference)
import functools

import jax
import jax.numpy as jnp
from jax import lax
from jax.experimental import pallas as pl
from jax.experimental.pallas import tpu as pltpu
from jax.experimental.pallas import tpu_sc as plsc

_V = 1000000
_D = 64
_R = 8
_LANES = 16
_NDB = _D // _LANES
_NW = 32
_BPW = 128
_H = 50


def _make_sc_kernel():
    mesh = plsc.VectorSubcoreMesh(core_axis_name="c", subcore_axis_name="s")

    @functools.partial(
        pl.kernel,
        mesh=mesh,
        compiler_params=pltpu.CompilerParams(use_tc_tiling_on_sc=False,
                                             needs_layout_passes=False),
        out_type=jax.ShapeDtypeStruct((_H, _D // 8, _NW, 8, _BPW),
                                      jnp.float32),
        scratch_types=[
            pltpu.VMEM((_H, _BPW), jnp.int32),
            pltpu.VMEM((_R, _BPW), jnp.int32),
            pltpu.VMEM((_R * _BPW,), jnp.float32),
            pltpu.VMEM((_BPW, _D), jnp.float32),
            pltpu.VMEM((_D // 8, 8, _BPW), jnp.float32),
            pltpu.VMEM((_R, _D), jnp.float32),
            pltpu.SemaphoreType.DMA,
        ],
    )
    def sc_kernel(idx_hbm, emb_hbm, at_hbm, b_hbm, out_hbm,
                  idx_v, ai_v, av_v, rows_v, tr_v, b_v, sem):
        num_cores = 2
        wid = lax.axis_index("s") * num_cores + lax.axis_index("c")

        pltpu.sync_copy(idx_hbm.at[:, pl.ds(wid * _BPW, _BPW)], idx_v)
        pltpu.sync_copy(b_hbm, b_v)

        b_vecs = [[b_v[r, pl.ds(db * _LANES, _LANES)] for db in range(_NDB)]
                  for r in range(_R)]
        g_vecs = [lax.iota(jnp.int32, _LANES) + g * _LANES
                  for g in range(_BPW // _LANES)]

        def unit_body(h, carry):
            for r in range(_R):
                for t in range(_BPW // _LANES):
                    sl = pl.ds(t * _LANES, _LANES)
                    ai_v[r, sl] = idx_v[h, sl] + r * _V
            cp_e = pltpu.async_copy(emb_hbm.at[idx_v.at[h]], rows_v, sem)
            cps_a = [pltpu.async_copy(at_hbm.at[ai_v.at[r]],
                                      av_v.at[pl.ds(r * _BPW, _BPW)], sem)
                     for r in range(_R)]
            cp_e.wait()
            for cp in cps_a:
                cp.wait()

            def row_body(k, c):
                avs = [plsc.load_gather(
                           av_v, [jnp.full((_LANES,), r * _BPW + k,
                                           jnp.int32)])
                       for r in range(_R)]
                for db in range(_NDB):
                    acc = rows_v[k, pl.ds(db * _LANES, _LANES)]
                    for r in range(_R):
                        acc = acc + avs[r] * b_vecs[r][db]
                    rows_v[k, pl.ds(db * _LANES, _LANES)] = acc
                return c

            lax.fori_loop(0, _BPW, row_body, 0)

            for j in range(_D):
                j_vec = jnp.full((_LANES,), j, jnp.int32)
                for g in range(_BPW // _LANES):
                    tv = plsc.load_gather(rows_v, [g_vecs[g], j_vec])
                    tr_v[j // 8, j % 8, pl.ds(g * _LANES, _LANES)] = tv
            pltpu.sync_copy(tr_v, out_hbm.at[h, :, wid])
            return carry

        lax.fori_loop(0, _H, unit_body, 0)

    return sc_kernel


_sc_kernel = _make_sc_kernel()


def kernel(inputs, embeddings, lora_A, lora_B):
    batch, hist = inputs.shape
    idx_t = inputs.T
    at_flat = lora_A.T.reshape(-1)
    b_scaled = lora_B * jnp.sqrt(jnp.asarray(_D, jnp.float32))
    out5 = _sc_kernel(idx_t, embeddings, at_flat, b_scaled)
    out = out5.transpose(2, 4, 0, 1, 3).reshape(batch, hist, _D)
    return out


# speedup vs baseline: 1.1371x; 1.1371x over previous
import functools

import jax
import jax.numpy as jnp
from jax import lax
from jax.experimental import pallas as pl
from jax.experimental.pallas import tpu as pltpu
from jax.experimental.pallas import tpu_sc as plsc

_V = 1000000
_D = 64
_R = 8
_LANES = 16
_NDB = _D // _LANES
_NW = 32
_BPW = 128
_H = 50


def _make_sc_kernel():
    mesh = plsc.VectorSubcoreMesh(core_axis_name="c", subcore_axis_name="s")

    @functools.partial(
        pl.kernel,
        mesh=mesh,
        compiler_params=pltpu.CompilerParams(use_tc_tiling_on_sc=False,
                                             needs_layout_passes=False),
        out_type=jax.ShapeDtypeStruct((_H, _D // 8, _NW, 8, _BPW),
                                      jnp.float32),
        scratch_types=[
            pltpu.VMEM((_H, _BPW), jnp.int32),
            pltpu.VMEM((_BPW, _R), jnp.float32),
            pltpu.VMEM((_BPW, _D), jnp.float32),
            pltpu.VMEM((_D // 8, 8, _BPW), jnp.float32),
            pltpu.VMEM((_R, _D), jnp.float32),
            pltpu.SemaphoreType.DMA,
        ],
    )
    def sc_kernel(idx_hbm, emb_hbm, a_hbm, b_hbm, out_hbm,
                  idx_v, av_v, rows_v, tr_v, b_v, sem):
        num_cores = 2
        wid = lax.axis_index("s") * num_cores + lax.axis_index("c")

        pltpu.sync_copy(idx_hbm.at[:, pl.ds(wid * _BPW, _BPW)], idx_v)
        pltpu.sync_copy(b_hbm, b_v)

        b_vecs = [[b_v[r, pl.ds(db * _LANES, _LANES)] for db in range(_NDB)]
                  for r in range(_R)]
        g_vecs = [lax.iota(jnp.int32, _LANES) + g * _LANES
                  for g in range(_BPW // _LANES)]

        def unit_body(h, carry):
            cp_e = pltpu.async_copy(emb_hbm.at[idx_v.at[h]], rows_v, sem)
            cp_a = pltpu.async_copy(a_hbm.at[idx_v.at[h]], av_v, sem)
            cp_e.wait()
            cp_a.wait()

            def row_body(k, c):
                k_vec = jnp.full((_LANES,), k, jnp.int32)
                avs = [plsc.load_gather(
                           av_v, [k_vec, jnp.full((_LANES,), r, jnp.int32)])
                       for r in range(_R)]
                for db in range(_NDB):
                    acc = rows_v[k, pl.ds(db * _LANES, _LANES)]
                    for r in range(_R):
                        acc = acc + avs[r] * b_vecs[r][db]
                    rows_v[k, pl.ds(db * _LANES, _LANES)] = acc
                return c

            lax.fori_loop(0, _BPW, row_body, 0)

            for j in range(_D):
                j_vec = jnp.full((_LANES,), j, jnp.int32)
                for g in range(_BPW // _LANES):
                    tv = plsc.load_gather(rows_v, [g_vecs[g], j_vec])
                    tr_v[j // 8, j % 8, pl.ds(g * _LANES, _LANES)] = tv
            pltpu.sync_copy(tr_v, out_hbm.at[h, :, wid])
            return carry

        lax.fori_loop(0, _H, unit_body, 0)

    return sc_kernel


_sc_kernel = _make_sc_kernel()


def kernel(inputs, embeddings, lora_A, lora_B):
    batch, hist = inputs.shape
    idx_t = inputs.T
    b_scaled = lora_B * jnp.sqrt(jnp.asarray(_D, jnp.float32))
    out5 = _sc_kernel(idx_t, embeddings, lora_A, b_scaled)
    out = out5.transpose(2, 4, 0, 1, 3).reshape(batch, hist, _D)
    return out


# speedup vs baseline: 1.2272x; 1.0792x over previous
import functools

import jax
import jax.numpy as jnp
from jax import lax
from jax.experimental import pallas as pl
from jax.experimental.pallas import tpu as pltpu
from jax.experimental.pallas import tpu_sc as plsc

_V = 1000000
_D = 64
_R = 8
_LANES = 16
_NDB = _D // _LANES
_NW = 32
_BPW = 128
_H = 50


def _make_sc_kernel():
    mesh = plsc.VectorSubcoreMesh(core_axis_name="c", subcore_axis_name="s")

    @functools.partial(
        pl.kernel,
        mesh=mesh,
        compiler_params=pltpu.CompilerParams(use_tc_tiling_on_sc=False,
                                             needs_layout_passes=False),
        out_type=jax.ShapeDtypeStruct((_H, _D // 8, _NW, 8, _BPW),
                                      jnp.float32),
        scratch_types=[
            pltpu.VMEM((_H, _BPW), jnp.int32),
            pltpu.VMEM((_BPW, _R), jnp.float32),
            pltpu.VMEM((_BPW, _D), jnp.float32),
            pltpu.VMEM((_D // 8, 8, _BPW + 1), jnp.float32),
            pltpu.VMEM((_R, _D), jnp.float32),
            pltpu.SemaphoreType.DMA,
        ],
    )
    def sc_kernel(idx_hbm, emb_hbm, a_hbm, b_hbm, out_hbm,
                  idx_v, av_v, rows_v, tr_v, b_v, sem):
        num_cores = 2
        wid = lax.axis_index("s") * num_cores + lax.axis_index("c")

        pltpu.sync_copy(idx_hbm.at[:, pl.ds(wid * _BPW, _BPW)], idx_v)
        pltpu.sync_copy(b_hbm, b_v)

        b_vecs = [[b_v[r, pl.ds(db * _LANES, _LANES)] for db in range(_NDB)]
                  for r in range(_R)]
        j_vecs = [lax.iota(jnp.int32, _LANES) + db * _LANES
                  for db in range(_NDB)]
        jt_vecs = [jv // 8 for jv in j_vecs]
        j8_vecs = [jv % 8 for jv in j_vecs]

        def unit_body(h, carry):
            cp_e = pltpu.async_copy(emb_hbm.at[idx_v.at[h]], rows_v, sem)
            cp_a = pltpu.async_copy(a_hbm.at[idx_v.at[h]], av_v, sem)
            cp_e.wait()
            cp_a.wait()

            def row_body(k, c):
                k_vec = jnp.full((_LANES,), k, jnp.int32)
                avs = [plsc.load_gather(
                           av_v, [k_vec, jnp.full((_LANES,), r, jnp.int32)])
                       for r in range(_R)]
                for db in range(_NDB):
                    acc = rows_v[k, pl.ds(db * _LANES, _LANES)]
                    for r in range(_R):
                        acc = acc + avs[r] * b_vecs[r][db]
                    plsc.store_scatter(
                        tr_v, [jt_vecs[db], j8_vecs[db], k_vec], acc)
                return c

            lax.fori_loop(0, _BPW, row_body, 0)
            pltpu.sync_copy(tr_v.at[:, :, pl.ds(0, _BPW)],
                            out_hbm.at[h, :, wid])
            return carry

        lax.fori_loop(0, _H, unit_body, 0)

    return sc_kernel


_sc_kernel = _make_sc_kernel()


def kernel(inputs, embeddings, lora_A, lora_B):
    batch, hist = inputs.shape
    idx_t = inputs.T
    b_scaled = lora_B * jnp.sqrt(jnp.asarray(_D, jnp.float32))
    out5 = _sc_kernel(idx_t, embeddings, lora_A, b_scaled)
    out = out5.transpose(2, 4, 0, 1, 3).reshape(batch, hist, _D)
    return out
